# baseline (speedup 1.0000x reference)
import functools

import jax
import jax.numpy as jnp
from jax import lax
from jax.experimental import pallas as pl
from jax.experimental.pallas import tpu as pltpu
from jax.experimental.pallas import tpu_sc as plsc

_VOCAB = 1000000
_D = 64
_P = 128
_B = 1024
_DOC_LEN = 200
_QUERY_LEN = 50

_NC = 2
_NS = 16
_NW = _NC * _NS
_BPW = _B // _NW
_DCH = 100
_DCHUNKS = _DOC_LEN // _DCH
_NDC = _BPW * _DCHUNKS
_DPAR_W = 112
_QPAR_W = 64
_NBUF = 4


def _pool_chunk(rows_ref, par_ref, chunk, n_rows, acc):
    def add_row(a, r, p):
        pv = jnp.full((16,), p, dtype=jnp.float32)
        a0, a1, a2, a3 = a
        lo0, hi0 = rows_ref[r, pl.ds(0, 16)], rows_ref[r, pl.ds(64, 16)]
        lo1, hi1 = rows_ref[r, pl.ds(16, 16)], rows_ref[r, pl.ds(80, 16)]
        lo2, hi2 = rows_ref[r, pl.ds(32, 16)], rows_ref[r, pl.ds(96, 16)]
        lo3, hi3 = rows_ref[r, pl.ds(48, 16)], rows_ref[r, pl.ds(112, 16)]
        a0 = a0 + (lo0 + pv * (hi0 - lo0))
        a1 = a1 + (lo1 + pv * (hi1 - lo1))
        a2 = a2 + (lo2 + pv * (hi2 - lo2))
        a3 = a3 + (lo3 + pv * (hi3 - lo3))
        return (a0, a1, a2, a3)

    ngroups = n_rows // 16
    rem = n_rows - 16 * ngroups

    def body(g, a):
        r0 = 16 * g
        pvec = par_ref[chunk, pl.ds(r0, 16)]
        for j in range(16):
            a = add_row(a, r0 + j, pvec[j])
        return a

    acc = lax.fori_loop(0, ngroups, body, acc, unroll=False)
    if rem:
        r0 = 16 * ngroups
        pvec = par_ref[chunk, pl.ds(r0, 16)]
        for j in range(rem):
            acc = add_row(acc, r0 + j, pvec[j])
    return acc


def _store_acc(acc_ref, i, acc):
    a0, a1, a2, a3 = acc
    acc_ref[i, pl.ds(0, 16)] = a0
    acc_ref[i, pl.ds(16, 16)] = a1
    acc_ref[i, pl.ds(32, 16)] = a2
    acc_ref[i, pl.ds(48, 16)] = a3


def _sc_pool_kernel(ddiv_hbm, dpar_hbm, qdiv_hbm, qpar_hbm, table_hbm,
                    d_out_hbm, q_out_hbm,
                    ddiv_v, dpar_v, qdiv_v, qpar_v,
                    rows0, rows1, rows2, rows3, dacc_v, qacc_v,
                    sem0, sem1, sem2, sem3):
    wid = lax.axis_index("s") * _NC + lax.axis_index("c")
    rows = (rows0, rows1, rows2, rows3)
    sems = (sem0, sem1, sem2, sem3)

    pltpu.sync_copy(ddiv_hbm.at[pl.ds(wid * _NDC, _NDC)], ddiv_v)
    pltpu.sync_copy(dpar_hbm.at[pl.ds(wid * _NDC, _NDC)], dpar_v)
    pltpu.sync_copy(qdiv_hbm.at[pl.ds(wid * _BPW, _BPW)], qdiv_v)
    pltpu.sync_copy(qpar_hbm.at[pl.ds(wid * _BPW, _BPW)], qpar_v)

    zero = jnp.zeros((16,), jnp.float32)
    z4 = (zero, zero, zero, zero)

    def d_start(chunk, b):
        return pltpu.async_copy(table_hbm.at[ddiv_v.at[chunk]], rows[b],
                                sems[b])

    def d_wait(chunk, b):
        pltpu.make_async_copy(table_hbm.at[ddiv_v.at[chunk]], rows[b],
                              sems[b]).wait()

    for b in range(_NBUF):
        d_start(b, b)

    def d_round(k, carry):
        acc = z4
        for b in range(_NBUF):
            chunk = _NBUF * k + b
            d_wait(chunk, b)
            acc = _pool_chunk(rows[b], dpar_v, chunk, _DCH, acc)
            if b % _DCHUNKS == _DCHUNKS - 1:
                _store_acc(dacc_v, 2 * k + b // _DCHUNKS, acc)
                acc = z4

            @pl.when(k < _NDC // _NBUF - 1)
            def _():
                d_start(chunk + _NBUF, b)

        return carry

    lax.fori_loop(0, _NDC // _NBUF, d_round, 0, unroll=False)
    pltpu.sync_copy(dacc_v, d_out_hbm.at[pl.ds(wid * _BPW, _BPW)])

    def q_start(i, b):
        return pltpu.async_copy(table_hbm.at[qdiv_v.at[i]],
                                rows[b].at[pl.ds(0, _QUERY_LEN)], sems[b])

    def q_wait(i, b):
        pltpu.make_async_copy(table_hbm.at[qdiv_v.at[i]],
                              rows[b].at[pl.ds(0, _QUERY_LEN)],
                              sems[b]).wait()

    for b in range(_NBUF):
        q_start(b, b)

    def q_round(k, carry):
        for b in range(_NBUF):
            i = _NBUF * k + b
            q_wait(i, b)
            acc = _pool_chunk(rows[b], qpar_v, i, _QUERY_LEN, z4)
            _store_acc(qacc_v, i, acc)

            @pl.when(k < _BPW // _NBUF - 1)
            def _():
                q_start(i + _NBUF, b)

        return carry

    lax.fori_loop(0, _BPW // _NBUF, q_round, 0, unroll=False)
    pltpu.sync_copy(qacc_v, q_out_hbm.at[pl.ds(wid * _BPW, _BPW)])


def _sc_pool(ddiv, dpar, qdiv, qpar, table2):
    mesh = plsc.VectorSubcoreMesh(core_axis_name="c", subcore_axis_name="s")
    fn = functools.partial(
        pl.kernel,
        mesh=mesh,
        out_type=[
            jax.ShapeDtypeStruct((_B, _D), jnp.float32),
            jax.ShapeDtypeStruct((_B, _D), jnp.float32),
        ],
        scratch_types=[
            pltpu.VMEM((_NDC, _DCH), jnp.int32),
            pltpu.VMEM((_NDC, _DPAR_W), jnp.float32),
            pltpu.VMEM((_BPW, _QUERY_LEN), jnp.int32),
            pltpu.VMEM((_BPW, _QPAR_W), jnp.float32),
            pltpu.VMEM((_DCH, _P), jnp.float32),
            pltpu.VMEM((_DCH, _P), jnp.float32),
            pltpu.VMEM((_DCH, _P), jnp.float32),
            pltpu.VMEM((_DCH, _P), jnp.float32),
            pltpu.VMEM((_BPW, _D), jnp.float32),
            pltpu.VMEM((_BPW, _D), jnp.float32),
            pltpu.SemaphoreType.DMA,
            pltpu.SemaphoreType.DMA,
            pltpu.SemaphoreType.DMA,
            pltpu.SemaphoreType.DMA,
        ],
    )(_sc_pool_kernel)
    return fn(ddiv, dpar, qdiv, qpar, table2)


def _tc_head_kernel(d_ref, q_ref, dw1_ref, db1_ref, dw2_ref, db2_ref,
                    qw1_ref, qb1_ref, qw2_ref, qb2_ref, out_ref):
    def dot_t(a, w):
        return lax.dot_general(a, w, (((1,), (1,)), ((), ())),
                               preferred_element_type=jnp.float32)

    d = d_ref[...] * (1.0 / _DOC_LEN)
    q = q_ref[...] * (1.0 / _QUERY_LEN)
    dh = jnp.maximum(dot_t(d, dw1_ref[...]) + db1_ref[...], 0.0)
    dp = dot_t(dh, dw2_ref[...]) + db2_ref[...]
    qh = jnp.maximum(dot_t(q, qw1_ref[...]) + qb1_ref[...], 0.0)
    qp = dot_t(qh, qw2_ref[...]) + qb2_ref[...]
    dn = jnp.maximum(jnp.sqrt(jnp.sum(dp * dp, axis=1, keepdims=True)), 1e-8)
    qn = jnp.maximum(jnp.sqrt(jnp.sum(qp * qp, axis=1, keepdims=True)), 1e-8)
    out_ref[...] = jnp.sum(dp * qp, axis=1, keepdims=True) / (dn * qn)


def _tc_head(d_sum, q_sum, d_w1, d_b1, d_w2, d_b2, q_w1, q_b1, q_w2, q_b2):
    return pl.pallas_call(
        _tc_head_kernel,
        out_shape=jax.ShapeDtypeStruct((_B, 1), jnp.float32),
    )(d_sum, q_sum, d_w1, d_b1.reshape(1, _P), d_w2, d_b2.reshape(1, _P),
      q_w1, q_b1.reshape(1, _D), q_w2, q_b2.reshape(1, _P))


def kernel(doc_ids, query_ids, table, d_w1, d_b1, d_w2, d_b2,
           q_w1, q_b1, q_w2, q_b2):
    doc_ids = doc_ids.astype(jnp.int32)
    query_ids = query_ids.astype(jnp.int32)
    table2 = table.reshape(_VOCAB // 2, 2 * _D)
    ddiv = (doc_ids >> 1).reshape(_B * _DCHUNKS, _DCH)
    dpar = jnp.pad((doc_ids & 1).astype(jnp.float32).reshape(
        _B * _DCHUNKS, _DCH), ((0, 0), (0, _DPAR_W - _DCH)))
    qdiv = query_ids >> 1
    qpar = jnp.pad((query_ids & 1).astype(jnp.float32),
                   ((0, 0), (0, _QPAR_W - _QUERY_LEN)))
    d_sum, q_sum = _sc_pool(ddiv, dpar, qdiv, qpar, table2)
    sim = _tc_head(d_sum, q_sum, d_w1, d_b1, d_w2, d_b2,
                   q_w1, q_b1, q_w2, q_b2)
    return sim.reshape(_B)


# speedup vs baseline: 1.3824x; 1.3824x over previous
import functools

import jax
import jax.numpy as jnp
from jax import lax
from jax.experimental import pallas as pl
from jax.experimental.pallas import tpu as pltpu
from jax.experimental.pallas import tpu_sc as plsc

_VOCAB = 1000000
_D = 64
_P = 128
_B = 1024
_DOC_LEN = 200
_QUERY_LEN = 50

_NC = 2
_NS = 16
_NW = _NC * _NS
_BPW = _B // _NW
_DCH = 100
_DCHUNKS = _DOC_LEN // _DCH
_NDC = _BPW * _DCHUNKS
_NBUF = 4


def _pool_chunk(rows_ref, n_rows, acc):
    def add_row(a, r):
        a0, a1, a2, a3 = a
        a0 = a0 + rows_ref[r, pl.ds(0, 16)]
        a1 = a1 + rows_ref[r, pl.ds(16, 16)]
        a2 = a2 + rows_ref[r, pl.ds(32, 16)]
        a3 = a3 + rows_ref[r, pl.ds(48, 16)]
        return (a0, a1, a2, a3)

    def body(j, a):
        r0 = 4 * j
        for k in range(4):
            a = add_row(a, r0 + k)
        return a

    acc = lax.fori_loop(0, n_rows // 4, body, acc, unroll=False)
    for r in range(n_rows - n_rows % 4, n_rows):
        acc = add_row(acc, r)
    return acc


def _store_acc(acc_ref, i, acc):
    a0, a1, a2, a3 = acc
    acc_ref[i, pl.ds(0, 16)] = a0
    acc_ref[i, pl.ds(16, 16)] = a1
    acc_ref[i, pl.ds(32, 16)] = a2
    acc_ref[i, pl.ds(48, 16)] = a3


def _sc_pool_kernel(didx_hbm, qidx_hbm, table_hbm, d_out_hbm, q_out_hbm,
                    didx_v, qidx_v, rows0, rows1, rows2, rows3,
                    dacc_v, qacc_v, sem0, sem1, sem2, sem3):
    wid = lax.axis_index("s") * _NC + lax.axis_index("c")
    rows = (rows0, rows1, rows2, rows3)
    sems = (sem0, sem1, sem2, sem3)

    pltpu.sync_copy(didx_hbm.at[pl.ds(wid * _NDC, _NDC)], didx_v)
    pltpu.sync_copy(qidx_hbm.at[pl.ds(wid * _BPW, _BPW)], qidx_v)

    zero = jnp.zeros((16,), jnp.float32)
    z4 = (zero, zero, zero, zero)

    def d_start(chunk, b):
        return pltpu.async_copy(table_hbm.at[didx_v.at[chunk]], rows[b],
                                sems[b])

    def d_wait(chunk, b):
        pltpu.make_async_copy(table_hbm.at[didx_v.at[chunk]], rows[b],
                              sems[b]).wait()

    for b in range(_NBUF):
        d_start(b, b)

    def d_round(k, carry):
        acc = z4
        for b in range(_NBUF):
            chunk = _NBUF * k + b
            d_wait(chunk, b)
            acc = _pool_chunk(rows[b], _DCH, acc)
            if b % _DCHUNKS == _DCHUNKS - 1:
                _store_acc(dacc_v, 2 * k + b // _DCHUNKS, acc)
                acc = z4

            @pl.when(k < _NDC // _NBUF - 1)
            def _():
                d_start(chunk + _NBUF, b)

        return carry

    lax.fori_loop(0, _NDC // _NBUF, d_round, 0, unroll=False)
    pltpu.sync_copy(dacc_v, d_out_hbm.at[pl.ds(wid * _BPW, _BPW)])

    def q_start(i, b):
        return pltpu.async_copy(table_hbm.at[qidx_v.at[i]],
                                rows[b].at[pl.ds(0, _QUERY_LEN)], sems[b])

    def q_wait(i, b):
        pltpu.make_async_copy(table_hbm.at[qidx_v.at[i]],
                              rows[b].at[pl.ds(0, _QUERY_LEN)],
                              sems[b]).wait()

    for b in range(_NBUF):
        q_start(b, b)

    def q_round(k, carry):
        for b in range(_NBUF):
            i = _NBUF * k + b
            q_wait(i, b)
            acc = _pool_chunk(rows[b], _QUERY_LEN, z4)
            _store_acc(qacc_v, i, acc)

            @pl.when(k < _BPW // _NBUF - 1)
            def _():
                q_start(i + _NBUF, b)

        return carry

    lax.fori_loop(0, _BPW // _NBUF, q_round, 0, unroll=False)
    pltpu.sync_copy(qacc_v, q_out_hbm.at[pl.ds(wid * _BPW, _BPW)])


def _sc_pool(didx, qidx, table2):
    mesh = plsc.VectorSubcoreMesh(core_axis_name="c", subcore_axis_name="s")
    fn = functools.partial(
        pl.kernel,
        mesh=mesh,
        out_type=[
            jax.ShapeDtypeStruct((_B, _D), jnp.float32),
            jax.ShapeDtypeStruct((_B, _D), jnp.float32),
        ],
        scratch_types=[
            pltpu.VMEM((_NDC, _DCH), jnp.int32),
            pltpu.VMEM((_BPW, _QUERY_LEN), jnp.int32),
            pltpu.VMEM((_DCH, _P), jnp.float32),
            pltpu.VMEM((_DCH, _P), jnp.float32),
            pltpu.VMEM((_DCH, _P), jnp.float32),
            pltpu.VMEM((_DCH, _P), jnp.float32),
            pltpu.VMEM((_BPW, _D), jnp.float32),
            pltpu.VMEM((_BPW, _D), jnp.float32),
            pltpu.SemaphoreType.DMA,
            pltpu.SemaphoreType.DMA,
            pltpu.SemaphoreType.DMA,
            pltpu.SemaphoreType.DMA,
        ],
    )(_sc_pool_kernel)
    return fn(didx, qidx, table2)


def _tc_head_kernel(d_ref, q_ref, dw1_ref, db1_ref, dw2_ref, db2_ref,
                    qw1_ref, qb1_ref, qw2_ref, qb2_ref, out_ref):
    def dot_t(a, w):
        return lax.dot_general(a, w, (((1,), (1,)), ((), ())),
                               preferred_element_type=jnp.float32)

    d = d_ref[...] * (1.0 / _DOC_LEN)
    q = q_ref[...] * (1.0 / _QUERY_LEN)
    dh = jnp.maximum(dot_t(d, dw1_ref[...]) + db1_ref[...], 0.0)
    dp = dot_t(dh, dw2_ref[...]) + db2_ref[...]
    qh = jnp.maximum(dot_t(q, qw1_ref[...]) + qb1_ref[...], 0.0)
    qp = dot_t(qh, qw2_ref[...]) + qb2_ref[...]
    dn = jnp.maximum(jnp.sqrt(jnp.sum(dp * dp, axis=1, keepdims=True)), 1e-8)
    qn = jnp.maximum(jnp.sqrt(jnp.sum(qp * qp, axis=1, keepdims=True)), 1e-8)
    out_ref[...] = jnp.sum(dp * qp, axis=1, keepdims=True) / (dn * qn)


def _tc_head(d_sum, q_sum, d_w1, d_b1, d_w2, d_b2, q_w1, q_b1, q_w2, q_b2):
    return pl.pallas_call(
        _tc_head_kernel,
        out_shape=jax.ShapeDtypeStruct((_B, 1), jnp.float32),
    )(d_sum, q_sum, d_w1, d_b1.reshape(1, _P), d_w2, d_b2.reshape(1, _P),
      q_w1, q_b1.reshape(1, _D), q_w2, q_b2.reshape(1, _P))


def kernel(doc_ids, query_ids, table, d_w1, d_b1, d_w2, d_b2,
           q_w1, q_b1, q_w2, q_b2):
    doc_ids = doc_ids.astype(jnp.int32)
    query_ids = query_ids.astype(jnp.int32)
    table2 = jnp.pad(table, ((0, 0), (0, _P - _D)))
    didx = doc_ids.reshape(_B * _DCHUNKS, _DCH)
    d_sum, q_sum = _sc_pool(didx, query_ids, table2)
    sim = _tc_head(d_sum, q_sum, d_w1, d_b1, d_w2, d_b2,
                   q_w1, q_b1, q_w2, q_b2)
    return sim.reshape(_B)
